# baseline (speedup 1.0000x reference)
import functools

import jax
import jax.numpy as jnp
from jax import lax
from jax.experimental import pallas as pl
from jax.experimental.pallas import tpu as pltpu
from jax.experimental.pallas import tpu_sc as plsc

_NUM_CORES = 2
_NUM_SUBCORES = 16
_NW = _NUM_CORES * _NUM_SUBCORES

_GPC = 8
_W = 104
_CHUNK = _GPC * _W


def _make_sc_gather(V, D, N):
  rows_per_w = N // _NW
  n_chunk = rows_per_w // _CHUNK
  assert rows_per_w % _CHUNK == 0 and N % _NW == 0

  mesh = plsc.VectorSubcoreMesh(core_axis_name="c", subcore_axis_name="s")

  @functools.partial(
      pl.kernel,
      mesh=mesh,
      compiler_params=pltpu.CompilerParams(use_tc_tiling_on_sc=False),
      out_type=jax.ShapeDtypeStruct((N, D), jnp.float32),
      scratch_types=[
          pltpu.VMEM((n_chunk, _GPC, _W), jnp.int32),
          pltpu.VMEM((2, _CHUNK, D), jnp.float32),
          pltpu.SemaphoreType.DMA,
      ],
  )
  def gather_kernel(x_hbm, table_hbm, out_hbm, idx_v, rows_v, gsem):
    wid = lax.axis_index("s") * _NUM_CORES + lax.axis_index("c")
    base = wid * rows_per_w

    pltpu.sync_copy(x_hbm.at[wid], idx_v)

    @pl.loop(0, n_chunk)
    def chunk_loop(c):
      b = lax.rem(c, 2)
      copies = []
      for g in range(_GPC):
        copies.append(
            pltpu.async_copy(
                table_hbm.at[idx_v.at[c, g]],
                rows_v.at[b, pl.ds(g * _W, _W)],
                gsem,
            ))

      @pl.when(c > 0)
      def _():
        pltpu.sync_copy(
            rows_v.at[1 - b],
            out_hbm.at[pl.ds(base + (c - 1) * _CHUNK, _CHUNK)])

      for cp in copies:
        cp.wait()

    pltpu.sync_copy(
        rows_v.at[(n_chunk - 1) % 2],
        out_hbm.at[pl.ds(base + (n_chunk - 1) * _CHUNK, _CHUNK)])

  return gather_kernel


def kernel(X, table):
  Bt, F = X.shape
  V, D = table.shape
  N = Bt * F
  zf = lax.optimization_barrier(jnp.float32(0.0))
  zi = lax.optimization_barrier(jnp.int32(0))
  x_flat = X.astype(jnp.int32).reshape(
      _NW, N // (_NW * _CHUNK), _GPC, _W) + zi
  out = _make_sc_gather(V, D, N)(x_flat, table + zf)
  return out.reshape(Bt, F, D) + zf


# speedup vs baseline: 1.6913x; 1.6913x over previous
import functools

import jax
import jax.numpy as jnp
from jax import lax
from jax.experimental import pallas as pl
from jax.experimental.pallas import tpu as pltpu
from jax.experimental.pallas import tpu_sc as plsc

_NUM_CORES = 2
_NUM_SUBCORES = 16
_NW = _NUM_CORES * _NUM_SUBCORES

_GPC = 8
_W = 104
_CHUNK = _GPC * _W


def _make_sc_gather(V, D, N):
  rows_per_w = N // _NW
  n_chunk = rows_per_w // _CHUNK
  assert rows_per_w % _CHUNK == 0 and N % _NW == 0

  mesh = plsc.VectorSubcoreMesh(core_axis_name="c", subcore_axis_name="s")

  @functools.partial(
      pl.kernel,
      mesh=mesh,
      compiler_params=pltpu.CompilerParams(use_tc_tiling_on_sc=False),
      out_type=jax.ShapeDtypeStruct((N, D), jnp.float32),
      scratch_types=[
          pltpu.VMEM((n_chunk, _GPC, _W), jnp.int32),
          pltpu.VMEM((2, _CHUNK, D), jnp.float32),
          pltpu.SemaphoreType.DMA,
      ],
  )
  def gather_kernel(x_hbm, table_hbm, out_hbm, idx_v, rows_v, gsem):
    wid = lax.axis_index("s") * _NUM_CORES + lax.axis_index("c")
    base = wid * rows_per_w

    pltpu.sync_copy(x_hbm.at[wid], idx_v)

    @pl.loop(0, n_chunk)
    def chunk_loop(c):
      b = lax.rem(c, 2)
      copies = []
      for g in range(_GPC):
        copies.append(
            pltpu.async_copy(
                table_hbm.at[idx_v.at[c, g]],
                rows_v.at[b, pl.ds(g * _W, _W)],
                gsem,
            ))

      @pl.when(c > 0)
      def _():
        pltpu.sync_copy(
            rows_v.at[1 - b],
            out_hbm.at[pl.ds(base + (c - 1) * _CHUNK, _CHUNK)])

      for cp in copies:
        cp.wait()

    pltpu.sync_copy(
        rows_v.at[(n_chunk - 1) % 2],
        out_hbm.at[pl.ds(base + (n_chunk - 1) * _CHUNK, _CHUNK)])

  return gather_kernel


def kernel(X, table):
  Bt, F = X.shape
  V, D = table.shape
  N = Bt * F
  x_flat = X.astype(jnp.int32).reshape(
      _NW, N // (_NW * _CHUNK), _GPC, _W)
  out = _make_sc_gather(V, D, N)(x_flat, table)
  return out.reshape(Bt, F, D)
